# Initial kernel scaffold; baseline (speedup 1.0000x reference)
import jax
import jax.numpy as jnp
from jax.experimental import pallas as pl


def kernel(contrast_map):
    raise NotImplementedError("write your pallas kernel here")


# speedup vs baseline: 81.6467x; 81.6467x over previous
import jax
import jax.numpy as jnp
from jax.experimental import pallas as pl
from jax.experimental.pallas import tpu as pltpu

TOPK = 5
KTHR = 26214
NEG = float("-inf")


def _image_kernel(x_ref, coords_ref, labels_ref):
    x = x_ref[0]
    H, W = x.shape
    col = jax.lax.broadcasted_iota(jnp.int32, (H, W), 1)
    row = jax.lax.broadcasted_iota(jnp.int32, (H, W), 0)

    def shift_cols(a, d):
        r = jnp.roll(a, -d, axis=1)
        valid = (col + d >= 0) & (col + d < W)
        return jnp.where(valid, r, NEG)

    def shift_rows(a, d):
        r = jnp.roll(a, -d, axis=0)
        valid = (row + d >= 0) & (row + d < H)
        return jnp.where(valid, r, NEG)

    m3 = jnp.maximum(jnp.maximum(shift_cols(x, -1), x), shift_cols(x, 1))
    m9 = jnp.maximum(jnp.maximum(shift_cols(m3, -3), m3), shift_cols(m3, 3))
    v3 = jnp.maximum(jnp.maximum(shift_rows(m9, -1), m9), shift_rows(m9, 1))
    lm = jnp.maximum(jnp.maximum(shift_rows(v3, -3), v3), shift_rows(v3, 3))

    peak = x == lm
    flat_idx = row * W + col
    masked = jnp.where(peak, x, NEG)

    big = jnp.int32(2**30)
    vals, idxs = [], []
    cur = masked
    for _ in range(TOPK):
        v = jnp.max(cur)
        idx = jnp.min(jnp.where(cur == v, flat_idx, big))
        vals.append(v)
        idxs.append(idx)
        cur = jnp.where(flat_idx == idx, NEG, cur)

    counts = [jnp.sum((x >= v).astype(jnp.int32)) for v in vals]

    v0 = vals[0]
    adaptive = 0.5 * v0
    valid = [(c < KTHR) & (v >= adaptive) for c, v in zip(counts, vals)]
    n_valid = jnp.maximum(
        1, sum(jnp.int32(0) + vi.astype(jnp.int32) for vi in valid))

    xs, ys, labels = [], [], []
    for j in range(TOPK):
        keep = jnp.int32(j) < n_valid
        fx = (idxs[j] % W).astype(jnp.float32)
        fy = (idxs[j] // W).astype(jnp.float32)
        xs.append(jnp.where(keep, fx, -1.0))
        ys.append(jnp.where(keep, fy, -1.0))
        labels.append(jnp.where(keep, 1.0, -1.0))

    coords = jnp.stack([jnp.stack(xs), jnp.stack(ys)], axis=-1)
    coords_ref[0] = coords
    labels_ref[0, 0] = jnp.stack(labels)


def kernel(contrast_map):
    B, C, H, W = contrast_map.shape
    x = contrast_map.reshape(B, H, W)
    coords, labels = pl.pallas_call(
        _image_kernel,
        grid=(B,),
        in_specs=[pl.BlockSpec((1, H, W), lambda i: (i, 0, 0))],
        out_specs=[
            pl.BlockSpec((1, TOPK, 2), lambda i: (i, 0, 0)),
            pl.BlockSpec((1, 1, TOPK), lambda i: (i, 0, 0)),
        ],
        out_shape=[
            jax.ShapeDtypeStruct((B, TOPK, 2), jnp.float32),
            jax.ShapeDtypeStruct((B, 1, TOPK), jnp.float32),
        ],
    )(x)
    return coords, labels.reshape(B, TOPK)
